# baseline (speedup 1.0000x reference)
import functools

import jax
import jax.numpy as jnp
from jax import lax
from jax.experimental import pallas as pl
from jax.experimental.pallas import tpu as pltpu
from jax.experimental.pallas import tpu_sc as plsc

_NC = 2
_NS = 16
_NW = _NC * _NS
_LANES = 16

_BLK = 128
_GCH = 128
_NBUF = 2


def _relayout_body(N, D, tt_hbm, outw_hbm, blk_bufs, stg_bufs, rsems, wsems):
    wid = lax.axis_index("s") * _NC + lax.axis_index("c")
    n_blocks = N // _BLK
    nblk_w = -(-n_blocks // _NW)
    wide_per_blk = _BLK // 2
    iota = lax.iota(jnp.int32, _LANES)
    d_vecs = [iota + d0 for d0 in range(0, D, _LANES)]

    def bid(i):
        return jnp.minimum(wid + i * _NW, n_blocks - 1)

    def fire_read(i, b):
        c0 = pl.multiple_of(bid(i) * _BLK, _BLK)
        pltpu.async_copy(tt_hbm.at[:, pl.ds(c0, _BLK)], blk_bufs[b], rsems[b])

    def transpose_block(b):
        blk_v, stg_v = blk_bufs[b], stg_bufs[b]
        for jj in range(_BLK):
            col = iota * 0 + jj
            for k, dv in enumerate(d_vecs):
                src = plsc.load_gather(blk_v, [dv, col])
                flat = jj * D + k * _LANES
                stg_v[flat // (2 * D), pl.ds(flat % (2 * D), _LANES)] = src

    def drain_read(b):
        pltpu.make_async_copy(tt_hbm.at[:, pl.ds(0, _BLK)], blk_bufs[b],
                              rsems[b]).wait()

    def drain_write(b):
        pltpu.make_async_copy(stg_bufs[b], outw_hbm.at[pl.ds(0, wide_per_blk)],
                              wsems[b]).wait()

    fire_read(0, 0)
    fire_read(1, 1)

    def pair(g, carry):
        for b in range(_NBUF):
            i = g * _NBUF + b
            drain_read(b)
            transpose_block(b)
            w0 = pl.multiple_of(bid(i) * wide_per_blk, 8)
            pltpu.async_copy(stg_bufs[b], outw_hbm.at[pl.ds(w0, wide_per_blk)],
                             wsems[b])
            drain_write(b)
            fire_read(i + _NBUF, b)
        return carry

    lax.fori_loop(0, nblk_w // _NBUF, pair, 0)
    drain_read(0)
    drain_read(1)


def _lookup_body(per_w, n_chunks, D, ids_hbm, off_hbm, tablew_hbm, out_hbm,
                 ids_buf, pb_buf, row_bufs, off_v, gsems, osems):
    wid = lax.axis_index("s") * _NC + lax.axis_index("c")
    pltpu.sync_copy(off_hbm, off_v)
    offv = off_v[...]
    rows_w = per_w // _GCH
    base_row = wid * rows_w
    start8 = pl.multiple_of((base_row >> 3) << 3, 8)
    skew = base_row - start8
    base_out = wid * per_w
    n_groups = _GCH // _LANES
    iota = lax.iota(jnp.int32, _LANES)

    pltpu.sync_copy(ids_hbm.at[pl.ds(start8, ids_buf.shape[0])], ids_buf)

    def prep_row(r, carry):
        for g in range(n_groups):
            sl = pl.ds(g * _LANES, _LANES)
            shifted = ids_buf[skew + r, sl] + offv
            ids_buf[skew + r, sl] = shifted >> 1
            pb_buf[r, sl] = (shifted & 1) * D
        return carry

    lax.fori_loop(0, rows_w, prep_row, 0)

    def fire_gather(ci, b):
        pltpu.async_copy(tablew_hbm.at[ids_buf.at[skew + ci]], row_bufs[b],
                         gsems[b])

    def parity_fix(ci, b):
        rows_v = row_bufs[b]
        for g in range(n_groups):
            jdiv = iota + g * _LANES
            pcol = pb_buf[ci, pl.ds(g * _LANES, _LANES)]
            for d in range(D):
                src = plsc.load_gather(rows_v, [jdiv, pcol + d])
                plsc.store_scatter(rows_v, [jdiv, iota * 0 + d], src)

    def drain_gather(b):
        pltpu.make_async_copy(tablew_hbm.at[pl.ds(0, _GCH)], row_bufs[b],
                              gsems[b]).wait()

    def drain_out(b):
        pltpu.make_async_copy(row_bufs[b], out_hbm.at[pl.ds(0, _GCH)],
                              osems[b]).wait()

    fire_gather(0, 0)
    fire_gather(1, 1)

    def pair(g, carry):
        for b in range(_NBUF):
            i = g * _NBUF + b
            drain_gather(b)
            parity_fix(i, b)
            o0 = pl.multiple_of(base_out + i * _GCH, _GCH)
            pltpu.async_copy(row_bufs[b], out_hbm.at[pl.ds(o0, _GCH)], osems[b])
            drain_out(b)
            fire_gather(jnp.minimum(i + _NBUF, n_chunks - 1), b)
        return carry

    lax.fori_loop(0, n_chunks // _NBUF, pair, 0)
    drain_gather(0)
    drain_gather(1)


def kernel(input_ids, table, offsets):
    B, L, H = input_ids.shape
    N, D = table.shape
    total = B * L * H
    per_w = total // _NW
    n_chunks = per_w // _GCH

    ids2 = input_ids.reshape(total // _GCH, _GCH)
    off16 = jnp.tile(offsets, _LANES // H).astype(jnp.int32)

    mesh = plsc.VectorSubcoreMesh(core_axis_name="c", subcore_axis_name="s")
    cparams = pltpu.CompilerParams(use_tc_tiling_on_sc=True,
                                   needs_layout_passes=False)

    relayout = functools.partial(
        pl.kernel,
        out_type=jax.ShapeDtypeStruct((N // 2, 2 * D), jnp.float32),
        mesh=mesh,
        compiler_params=cparams,
        scratch_types=[
            [pltpu.VMEM((D, _BLK), jnp.float32) for _ in range(_NBUF)],
            [pltpu.VMEM((_BLK // 2, 2 * D), jnp.float32) for _ in range(_NBUF)],
            [pltpu.SemaphoreType.DMA for _ in range(_NBUF)],
            [pltpu.SemaphoreType.DMA for _ in range(_NBUF)],
        ],
    )(functools.partial(_relayout_body, N, D))
    tablew = relayout(table.T)

    lookup = functools.partial(
        pl.kernel,
        out_type=jax.ShapeDtypeStruct((total, 2 * D), jnp.float32),
        mesh=mesh,
        compiler_params=cparams,
        scratch_types=[
            pltpu.VMEM((per_w // _GCH + 4, _GCH), jnp.int32),
            pltpu.VMEM((per_w // _GCH, _GCH), jnp.int32),
            [pltpu.VMEM((_GCH, 2 * D), jnp.float32) for _ in range(_NBUF)],
            pltpu.VMEM((_LANES,), jnp.int32),
            [pltpu.SemaphoreType.DMA for _ in range(_NBUF)],
            [pltpu.SemaphoreType.DMA for _ in range(_NBUF)],
        ],
    )(functools.partial(_lookup_body, per_w, n_chunks, D))
    out = lookup(ids2, off16, tablew)
    return out[:, :D].reshape(B, L, H, D)


# speedup vs baseline: 2.9261x; 2.9261x over previous
import functools

import jax
import jax.numpy as jnp
from jax import lax
from jax.experimental import pallas as pl
from jax.experimental.pallas import tpu as pltpu
from jax.experimental.pallas import tpu_sc as plsc

_NC = 2
_NS = 16
_NW = _NC * _NS
_LANES = 16

_BLK = 128
_GCH = 128
_NBUF = 2


def _relayout_body(N, D, tt_hbm, outw_hbm, blk_bufs, stg_bufs, rsems, wsems):
    wid = lax.axis_index("s") * _NC + lax.axis_index("c")
    n_blocks = N // _BLK
    nblk_w = -(-n_blocks // _NW)
    wide_per_blk = _BLK // 2
    iota = lax.iota(jnp.int32, _LANES)
    d_vecs = [iota + d0 for d0 in range(0, D, _LANES)]

    def bid(i):
        return jnp.minimum(wid + i * _NW, n_blocks - 1)

    def fire_read(i, b):
        c0 = pl.multiple_of(bid(i) * _BLK, _BLK)
        pltpu.async_copy(tt_hbm.at[:, pl.ds(c0, _BLK)], blk_bufs[b], rsems[b])

    rot_vecs = [(iota + k) & (_LANES - 1) for k in range(_LANES)]
    rot64_vecs = [r * D + iota for r in rot_vecs]

    def transpose_block(b):
        blk_v, stg_v = blk_bufs[b], stg_bufs[b]

        def jt_step(jt, carry):
            jj0 = jt * _LANES
            for dt, dv in enumerate(d_vecs):
                d0 = dt * _LANES
                for k in range(_LANES):
                    src = plsc.load_gather(blk_v, [dv, jj0 + rot_vecs[k]])
                    flat = rot64_vecs[k] + (jj0 * D + d0)
                    plsc.store_scatter(
                        stg_v, [flat >> 7, flat & (2 * D - 1)], src)
            return carry

        lax.fori_loop(0, _BLK // _LANES, jt_step, 0)

    def drain_read(b):
        pltpu.make_async_copy(tt_hbm.at[:, pl.ds(0, _BLK)], blk_bufs[b],
                              rsems[b]).wait()

    def drain_write(b):
        pltpu.make_async_copy(stg_bufs[b], outw_hbm.at[pl.ds(0, wide_per_blk)],
                              wsems[b]).wait()

    fire_read(0, 0)
    fire_read(1, 1)

    def pair(g, carry):
        for b in range(_NBUF):
            i = g * _NBUF + b
            drain_read(b)
            transpose_block(b)
            w0 = pl.multiple_of(bid(i) * wide_per_blk, 8)
            pltpu.async_copy(stg_bufs[b], outw_hbm.at[pl.ds(w0, wide_per_blk)],
                             wsems[b])
            drain_write(b)
            fire_read(i + _NBUF, b)
        return carry

    lax.fori_loop(0, nblk_w // _NBUF, pair, 0)
    drain_read(0)
    drain_read(1)


def _lookup_body(per_w, n_chunks, D, ids_hbm, off_hbm, tablew_hbm, out_hbm,
                 ids_buf, pb_buf, row_bufs, off_v, gsems, osems):
    wid = lax.axis_index("s") * _NC + lax.axis_index("c")
    pltpu.sync_copy(off_hbm, off_v)
    offv = off_v[...]
    rows_w = per_w // _GCH
    base_row = wid * rows_w
    start8 = pl.multiple_of((base_row >> 3) << 3, 8)
    skew = base_row - start8
    base_out = wid * per_w
    n_groups = _GCH // _LANES
    iota = lax.iota(jnp.int32, _LANES)

    pltpu.sync_copy(ids_hbm.at[pl.ds(start8, ids_buf.shape[0])], ids_buf)

    def prep_row(r, carry):
        for g in range(n_groups):
            sl = pl.ds(g * _LANES, _LANES)
            shifted = ids_buf[skew + r, sl] + offv
            ids_buf[skew + r, sl] = shifted >> 1
            pb_buf[r, sl] = (shifted & 1) * D
        return carry

    lax.fori_loop(0, rows_w, prep_row, 0)

    def fire_gather(ci, b):
        pltpu.async_copy(tablew_hbm.at[ids_buf.at[skew + ci]], row_bufs[b],
                         gsems[b])

    rot_vecs = [(iota + k) & (_LANES - 1) for k in range(_LANES)]

    def parity_fix(ci, b):
        rows_v = row_bufs[b]

        def g_step(g, carry):
            jdiv = iota + g * _LANES
            pcol = pb_buf[ci, pl.ds(g * _LANES, _LANES)]
            for dt in range(D // _LANES):
                base = pcol + dt * _LANES
                for k in range(_LANES):
                    drot = rot_vecs[k] + dt * _LANES
                    src = plsc.load_gather(rows_v, [jdiv, base + rot_vecs[k]])
                    plsc.store_scatter(rows_v, [jdiv, drot], src)
            return carry

        lax.fori_loop(0, n_groups, g_step, 0)

    def drain_gather(b):
        pltpu.make_async_copy(tablew_hbm.at[pl.ds(0, _GCH)], row_bufs[b],
                              gsems[b]).wait()

    def drain_out(b):
        pltpu.make_async_copy(row_bufs[b], out_hbm.at[pl.ds(0, _GCH)],
                              osems[b]).wait()

    fire_gather(0, 0)
    fire_gather(1, 1)

    def pair(g, carry):
        for b in range(_NBUF):
            i = g * _NBUF + b
            drain_gather(b)
            parity_fix(i, b)
            o0 = pl.multiple_of(base_out + i * _GCH, _GCH)
            pltpu.async_copy(row_bufs[b], out_hbm.at[pl.ds(o0, _GCH)], osems[b])
            drain_out(b)
            fire_gather(jnp.minimum(i + _NBUF, n_chunks - 1), b)
        return carry

    lax.fori_loop(0, n_chunks // _NBUF, pair, 0)
    drain_gather(0)
    drain_gather(1)


def kernel(input_ids, table, offsets):
    B, L, H = input_ids.shape
    N, D = table.shape
    total = B * L * H
    per_w = total // _NW
    n_chunks = per_w // _GCH

    ids2 = input_ids.reshape(total // _GCH, _GCH)
    off16 = jnp.tile(offsets, _LANES // H).astype(jnp.int32)

    mesh = plsc.VectorSubcoreMesh(core_axis_name="c", subcore_axis_name="s")
    cparams = pltpu.CompilerParams(use_tc_tiling_on_sc=True,
                                   needs_layout_passes=False)

    relayout = functools.partial(
        pl.kernel,
        out_type=jax.ShapeDtypeStruct((N // 2, 2 * D), jnp.float32),
        mesh=mesh,
        compiler_params=cparams,
        scratch_types=[
            [pltpu.VMEM((D, _BLK), jnp.float32) for _ in range(_NBUF)],
            [pltpu.VMEM((_BLK // 2, 2 * D), jnp.float32) for _ in range(_NBUF)],
            [pltpu.SemaphoreType.DMA for _ in range(_NBUF)],
            [pltpu.SemaphoreType.DMA for _ in range(_NBUF)],
        ],
    )(functools.partial(_relayout_body, N, D))
    tablew = relayout(table.T)

    lookup = functools.partial(
        pl.kernel,
        out_type=jax.ShapeDtypeStruct((total, 2 * D), jnp.float32),
        mesh=mesh,
        compiler_params=cparams,
        scratch_types=[
            pltpu.VMEM((per_w // _GCH + 4, _GCH), jnp.int32),
            pltpu.VMEM((per_w // _GCH, _GCH), jnp.int32),
            [pltpu.VMEM((_GCH, 2 * D), jnp.float32) for _ in range(_NBUF)],
            pltpu.VMEM((_LANES,), jnp.int32),
            [pltpu.SemaphoreType.DMA for _ in range(_NBUF)],
            [pltpu.SemaphoreType.DMA for _ in range(_NBUF)],
        ],
    )(functools.partial(_lookup_body, per_w, n_chunks, D))
    out = lookup(ids2, off16, tablew)
    return out[:, :D].reshape(B, L, H, D)


# speedup vs baseline: 3.3731x; 1.1528x over previous
import functools

import jax
import jax.numpy as jnp
from jax import lax
from jax.experimental import pallas as pl
from jax.experimental.pallas import tpu as pltpu
from jax.experimental.pallas import tpu_sc as plsc

_NC = 2
_NS = 16
_NW = _NC * _NS
_LANES = 16

_BLK = 128
_GCH = 128
_NBUF = 2


def _relayout_body(N, D, tt_hbm, outw_hbm, blk_bufs, stg_bufs, rsems, wsems):
    wid = lax.axis_index("s") * _NC + lax.axis_index("c")
    n_blocks = N // _BLK
    nblk_w = -(-n_blocks // _NW)
    wide_per_blk = _BLK // 2
    iota = lax.iota(jnp.int32, _LANES)
    d_vecs = [iota + d0 for d0 in range(0, D, _LANES)]

    def bid(i):
        return jnp.minimum(wid + i * _NW, n_blocks - 1)

    def fire_read(i, b):
        c0 = pl.multiple_of(bid(i) * _BLK, _BLK)
        pltpu.async_copy(tt_hbm.at[:, pl.ds(c0, _BLK)], blk_bufs[b], rsems[b])

    rot_vecs = [(iota + k) & (_LANES - 1) for k in range(_LANES)]
    col_vecs = [[dv + ((iota + k) & 1) * D for k in range(2)] for dv in d_vecs]

    def transpose_block(b):
        blk_v, stg_v = blk_bufs[b], stg_bufs[b]

        def jt_step(jt, carry):
            jj0 = jt * _LANES
            for k in range(_LANES):
                jr = jj0 + rot_vecs[k]
                jrh = jr >> 1
                for dt, dv in enumerate(d_vecs):
                    src = plsc.load_gather(blk_v, [dv, jr])
                    plsc.store_scatter(stg_v, [jrh, col_vecs[dt][k & 1]], src)
            return carry

        lax.fori_loop(0, _BLK // _LANES, jt_step, 0)

    def drain_read(b):
        pltpu.make_async_copy(tt_hbm.at[:, pl.ds(0, _BLK)], blk_bufs[b],
                              rsems[b]).wait()

    def drain_write(b):
        pltpu.make_async_copy(stg_bufs[b], outw_hbm.at[pl.ds(0, wide_per_blk)],
                              wsems[b]).wait()

    def fire_write(i, b):
        w0 = pl.multiple_of(bid(i) * wide_per_blk, 8)
        pltpu.async_copy(stg_bufs[b], outw_hbm.at[pl.ds(w0, wide_per_blk)],
                         wsems[b])

    fire_read(0, 0)
    fire_read(1, 1)
    for b in range(_NBUF):
        drain_read(b)
        transpose_block(b)
        fire_write(b, b)
        fire_read(b + _NBUF, b)

    def pair(g, carry):
        for b in range(_NBUF):
            i = g * _NBUF + b
            drain_read(b)
            drain_write(b)
            transpose_block(b)
            fire_write(i, b)
            fire_read(i + _NBUF, b)
        return carry

    lax.fori_loop(1, nblk_w // _NBUF, pair, 0)
    drain_read(0)
    drain_read(1)
    drain_write(0)
    drain_write(1)


def _lookup_body(per_w, n_chunks, D, ids_hbm, off_hbm, tablew_hbm, out_hbm,
                 ids_buf, pb_buf, row_bufs, sel_bufs, off_v, gsems, osems):
    wid = lax.axis_index("s") * _NC + lax.axis_index("c")
    pltpu.sync_copy(off_hbm, off_v)
    offv = off_v[...]
    rows_w = per_w // _GCH
    base_row = wid * rows_w
    start8 = pl.multiple_of((base_row >> 3) << 3, 8)
    skew = base_row - start8
    base_out = wid * per_w
    n_groups = _GCH // _LANES
    iota = lax.iota(jnp.int32, _LANES)

    pltpu.sync_copy(ids_hbm.at[pl.ds(start8, ids_buf.shape[0])], ids_buf)

    def prep_row(r, carry):
        for g in range(n_groups):
            sl = pl.ds(g * _LANES, _LANES)
            shifted = ids_buf[skew + r, sl] + offv
            ids_buf[skew + r, sl] = shifted >> 1
            pb_buf[r, sl] = (shifted & 1) * D
        return carry

    lax.fori_loop(0, rows_w, prep_row, 0)

    def fire_gather(ci, b):
        pltpu.async_copy(tablew_hbm.at[ids_buf.at[skew + ci]], row_bufs[b],
                         gsems[b])

    rot_vecs = [(iota + k) & (_LANES - 1) for k in range(_LANES)]
    drot_vecs = [[rot_vecs[k] + dt * _LANES for k in range(_LANES)]
                 for dt in range(D // _LANES)]

    def parity_fix(ci, b):
        rows_v, sel_v = row_bufs[b], sel_bufs[b]

        def g_step(g, carry):
            jdiv = iota + g * _LANES
            pcol = pb_buf[ci, pl.ds(g * _LANES, _LANES)]
            for dt in range(D // _LANES):
                base = pcol + dt * _LANES
                for k in range(_LANES):
                    src = plsc.load_gather(rows_v, [jdiv, base + rot_vecs[k]])
                    plsc.store_scatter(sel_v, [jdiv, drot_vecs[dt][k]], src)
            return carry

        lax.fori_loop(0, n_groups, g_step, 0)

    def drain_gather(b):
        pltpu.make_async_copy(tablew_hbm.at[pl.ds(0, _GCH)], row_bufs[b],
                              gsems[b]).wait()

    def drain_out(b):
        pltpu.make_async_copy(sel_bufs[b], out_hbm.at[pl.ds(0, _GCH)],
                              osems[b]).wait()

    def fire_out(i, b):
        o0 = pl.multiple_of(base_out + i * _GCH, _GCH)
        pltpu.async_copy(sel_bufs[b], out_hbm.at[pl.ds(o0, _GCH)], osems[b])

    fire_gather(0, 0)
    fire_gather(1, 1)
    for b in range(_NBUF):
        drain_gather(b)
        parity_fix(b, b)
        fire_out(b, b)
        fire_gather(b + _NBUF, b)

    def pair(g, carry):
        for b in range(_NBUF):
            i = g * _NBUF + b
            drain_gather(b)
            drain_out(b)
            parity_fix(i, b)
            fire_gather(jnp.minimum(i + _NBUF, n_chunks - 1), b)
            fire_out(i, b)
        return carry

    lax.fori_loop(1, n_chunks // _NBUF, pair, 0)
    drain_gather(0)
    drain_gather(1)
    drain_out(0)
    drain_out(1)


def kernel(input_ids, table, offsets):
    B, L, H = input_ids.shape
    N, D = table.shape
    total = B * L * H
    per_w = total // _NW
    n_chunks = per_w // _GCH

    ids2 = input_ids.reshape(total // _GCH, _GCH)
    off16 = jnp.tile(offsets, _LANES // H).astype(jnp.int32)

    mesh = plsc.VectorSubcoreMesh(core_axis_name="c", subcore_axis_name="s")
    cparams = pltpu.CompilerParams(use_tc_tiling_on_sc=True,
                                   needs_layout_passes=False)

    relayout = functools.partial(
        pl.kernel,
        out_type=jax.ShapeDtypeStruct((N // 2, 2 * D), jnp.float32),
        mesh=mesh,
        compiler_params=cparams,
        scratch_types=[
            [pltpu.VMEM((D, _BLK), jnp.float32) for _ in range(_NBUF)],
            [pltpu.VMEM((_BLK // 2, 2 * D), jnp.float32) for _ in range(_NBUF)],
            [pltpu.SemaphoreType.DMA for _ in range(_NBUF)],
            [pltpu.SemaphoreType.DMA for _ in range(_NBUF)],
        ],
    )(functools.partial(_relayout_body, N, D))
    tablew = relayout(table.T)

    lookup = functools.partial(
        pl.kernel,
        out_type=jax.ShapeDtypeStruct((total, 2 * D), jnp.float32),
        mesh=mesh,
        compiler_params=cparams,
        scratch_types=[
            pltpu.VMEM((per_w // _GCH + 4, _GCH), jnp.int32),
            pltpu.VMEM((per_w // _GCH, _GCH), jnp.int32),
            [pltpu.VMEM((_GCH, 2 * D), jnp.float32) for _ in range(_NBUF)],
            [pltpu.VMEM((_GCH, 2 * D), jnp.float32) for _ in range(_NBUF)],
            pltpu.VMEM((_LANES,), jnp.int32),
            [pltpu.SemaphoreType.DMA for _ in range(_NBUF)],
            [pltpu.SemaphoreType.DMA for _ in range(_NBUF)],
        ],
    )(functools.partial(_lookup_body, per_w, n_chunks, D))
    out = lookup(ids2, off16, tablew)
    return out[:, :D].reshape(B, L, H, D)


# speedup vs baseline: 3.6087x; 1.0698x over previous
import functools

import jax
import jax.numpy as jnp
from jax import lax
from jax.experimental import pallas as pl
from jax.experimental.pallas import tpu as pltpu
from jax.experimental.pallas import tpu_sc as plsc

_NC = 2
_NS = 16
_NW = _NC * _NS
_LANES = 16

_GCH = 128
_NBUF = 3


def _lookup_body(per_w, n_chunks, D, ids_hbm, off_hbm, tablep_hbm, out_hbm,
                 ids_buf, row_bufs, off_v, gsems, osems):
    wid = lax.axis_index("s") * _NC + lax.axis_index("c")
    pltpu.sync_copy(off_hbm, off_v)
    offv = off_v[...]
    rows_w = per_w // _GCH
    base_row = wid * rows_w
    start8 = pl.multiple_of((base_row >> 3) << 3, 8)
    skew = base_row - start8
    base_out = wid * per_w
    n_groups = _GCH // _LANES

    pltpu.sync_copy(ids_hbm.at[pl.ds(start8, ids_buf.shape[0])], ids_buf)

    def prep_row(r, carry):
        for g in range(n_groups):
            sl = pl.ds(g * _LANES, _LANES)
            ids_buf[skew + r, sl] = ids_buf[skew + r, sl] + offv
        return carry

    lax.fori_loop(0, rows_w, prep_row, 0)

    def fire_gather(ci, b):
        pltpu.async_copy(tablep_hbm.at[ids_buf.at[skew + ci]], row_bufs[b],
                         gsems[b])

    def fire_out(ci, b):
        o0 = pl.multiple_of(base_out + ci * _GCH, _GCH)
        pltpu.async_copy(row_bufs[b], out_hbm.at[pl.ds(o0, _GCH)], osems[b])

    def drain_gather(b):
        pltpu.make_async_copy(tablep_hbm.at[pl.ds(0, _GCH)], row_bufs[b],
                              gsems[b]).wait()

    def drain_out(b):
        pltpu.make_async_copy(row_bufs[b], out_hbm.at[pl.ds(0, _GCH)],
                              osems[b]).wait()

    fire_gather(0, 0)
    fire_gather(1, 1)
    fire_gather(2, 2)
    for i in range(3):
        drain_gather(i % _NBUF)
        fire_out(i, i % _NBUF)
    drain_out(0)
    fire_gather(3, 0)

    def triple(t, carry):
        for u in range(_NBUF):
            b = u
            i = 3 * t + 3 + u
            drain_gather(b)
            fire_out(i, b)
            drain_out((b + 1) % _NBUF)
            fire_gather(i + 1, (u + 1) % _NBUF)
        return carry

    lax.fori_loop(0, (n_chunks - 4) // _NBUF, triple, 0)
    i_last = n_chunks - 1
    drain_gather(i_last % _NBUF)
    fire_out(i_last, i_last % _NBUF)
    drain_out((i_last + 1) % _NBUF)
    drain_out((i_last + 2) % _NBUF)
    drain_out(i_last % _NBUF)


def kernel(input_ids, table, offsets):
    B, L, H = input_ids.shape
    N, D = table.shape
    total = B * L * H
    per_w = total // _NW
    n_chunks = per_w // _GCH

    ids2 = input_ids.reshape(total // _GCH, _GCH)
    tablep = jnp.pad(table, ((0, 0), (0, D)))
    off16 = jnp.tile(offsets, _LANES // H).astype(jnp.int32)

    mesh = plsc.VectorSubcoreMesh(core_axis_name="c", subcore_axis_name="s")
    cparams = pltpu.CompilerParams(use_tc_tiling_on_sc=True,
                                   needs_layout_passes=False)

    lookup = functools.partial(
        pl.kernel,
        out_type=jax.ShapeDtypeStruct((total, 2 * D), jnp.float32),
        mesh=mesh,
        compiler_params=cparams,
        scratch_types=[
            pltpu.VMEM((per_w // _GCH + 4, _GCH), jnp.int32),
            [pltpu.VMEM((_GCH, 2 * D), jnp.float32) for _ in range(_NBUF)],
            pltpu.VMEM((_LANES,), jnp.int32),
            [pltpu.SemaphoreType.DMA for _ in range(_NBUF)],
            [pltpu.SemaphoreType.DMA for _ in range(_NBUF)],
        ],
    )(functools.partial(_lookup_body, per_w, n_chunks, D))
    out = lookup(ids2, off16, tablep)
    return out[:, :D].reshape(B, L, H, D)


# speedup vs baseline: 3.7272x; 1.0328x over previous
import functools

import jax
import jax.numpy as jnp
from jax import lax
from jax.experimental import pallas as pl
from jax.experimental.pallas import tpu as pltpu
from jax.experimental.pallas import tpu_sc as plsc

_NC = 2
_NS = 16
_NW = _NC * _NS
_LANES = 16

_GCH = 128
_CHUNK = 256
_NBUF = 2


def _body(per_w, n_chunks, ids_hbm, off_hbm, tablep_hbm, out_hbm,
          idx_bufs, row_bufs, off_v, gsems, osems):
    wid = lax.axis_index("s") * _NC + lax.axis_index("c")
    pltpu.sync_copy(off_hbm, off_v)
    offv = off_v[...]
    idx_rows = _CHUNK // _GCH
    base_idx_row = wid * (per_w // _GCH)
    base_out = wid * per_w

    def load_and_fire(ci, b):
        idx_v, rows_v = idx_bufs[b], row_bufs[b]
        pltpu.sync_copy(ids_hbm.at[pl.ds(base_idx_row + ci * idx_rows, idx_rows)],
                        idx_v)
        for r in range(idx_rows):
            for c in range(_GCH // _LANES):
                sl = pl.ds(c * _LANES, _LANES)
                idx_v[r, sl] = idx_v[r, sl] + offv
        return [
            pltpu.async_copy(tablep_hbm.at[idx_v.at[r]],
                             rows_v.at[pl.ds(r * _GCH, _GCH)], gsems[b])
            for r in range(idx_rows)
        ]

    gcopies = {0: load_and_fire(0, 0)}
    ocopies = {}
    for ci in range(n_chunks):
        b = ci % _NBUF
        for cp in gcopies.pop(ci):
            cp.wait()
        ocopies[ci] = pltpu.async_copy(
            row_bufs[b], out_hbm.at[pl.ds(base_out + ci * _CHUNK, _CHUNK)],
            osems[b])
        if ci + 1 < n_chunks:
            nb = (ci + 1) % _NBUF
            if ci >= 1:
                ocopies.pop(ci - 1).wait()
            gcopies[ci + 1] = load_and_fire(ci + 1, nb)
    for ci in list(ocopies):
        ocopies.pop(ci).wait()


def kernel(input_ids, table, offsets):
    B, L, H = input_ids.shape
    N, D = table.shape
    total = B * L * H
    per_w = total // _NW
    n_chunks = per_w // _CHUNK

    ids2 = input_ids.reshape(total // _GCH, _GCH)
    tablep = jnp.pad(table, ((0, 0), (0, D)))
    off16 = jnp.tile(offsets, _LANES // H).astype(jnp.int32)

    mesh = plsc.VectorSubcoreMesh(core_axis_name="c", subcore_axis_name="s")
    run = functools.partial(
        pl.kernel,
        out_type=jax.ShapeDtypeStruct((total, 2 * D), jnp.float32),
        mesh=mesh,
        compiler_params=pltpu.CompilerParams(use_tc_tiling_on_sc=True),
        scratch_types=[
            [pltpu.VMEM((_CHUNK // _GCH, _GCH), jnp.int32) for _ in range(_NBUF)],
            [pltpu.VMEM((_CHUNK, 2 * D), jnp.float32) for _ in range(_NBUF)],
            pltpu.VMEM((_LANES,), jnp.int32),
            [pltpu.SemaphoreType.DMA for _ in range(_NBUF)],
            [pltpu.SemaphoreType.DMA for _ in range(_NBUF)],
        ],
    )(functools.partial(_body, per_w, n_chunks))
    out = run(ids2, off16, tablep)
    return out[:, :D].reshape(B, L, H, D)
